# baseline (speedup 1.0000x reference)
import functools

import jax
import jax.numpy as jnp
from jax import lax
from jax.experimental import pallas as pl
from jax.experimental.pallas import tpu as pltpu, tpu_sc as plsc

N = 10000
E = 320000
R = 4
RN = R * N
D = 128
NC = 2
NS = 16
LW = 128
EPAD = 327680
EROWS = EPAD // LW
ROWS_PER_SUB = EROWS // NS
ROWS_PER_WRK = EROWS // (NC * NS)
RNP = 40960
NSLICE = RNP // NS
NBUF = 2

_mesh = plsc.VectorSubcoreMesh(core_axis_name="c", subcore_axis_name="s")
_sc_params = pltpu.CompilerParams(use_tc_tiling_on_sc=False)


@functools.partial(
    pl.kernel,
    out_type=jax.ShapeDtypeStruct((NC, RNP, 8), jnp.float32),
    mesh=_mesh,
    scratch_types=[
        pltpu.VMEM((ROWS_PER_WRK, LW), jnp.int32),
        pltpu.VMEM((LW, 8), jnp.float32),
        pltpu.VMEM_SHARED((RNP, 8), jnp.float32),
        pltpu.SemaphoreType.DMA,
    ],
    compiler_params=_sc_params,
)
def _deg_kernel(dsti_hbm, ones_hbm, zeros_hbm, out_hbm, dsti_v, ones_v, acc_sh,
                sem):
    c = lax.axis_index("c")
    s = lax.axis_index("s")
    wid = s * NC + c
    pltpu.sync_copy(zeros_hbm.at[pl.ds(s * NSLICE, NSLICE)],
                    acc_sh.at[pl.ds(s * NSLICE, NSLICE)])
    pltpu.sync_copy(dsti_hbm.at[pl.ds(wid * ROWS_PER_WRK, ROWS_PER_WRK)], dsti_v)
    pltpu.sync_copy(ones_hbm, ones_v)
    plsc.subcore_barrier()

    def fire(j, carry):
        pltpu.async_copy(ones_v, acc_sh.at[dsti_v.at[j]], sem, add=True)
        return carry

    def drain(j, carry):
        pltpu.make_async_copy(ones_v, acc_sh.at[dsti_v.at[j]], sem).wait()
        return carry

    lax.fori_loop(0, ROWS_PER_WRK, fire, 0, unroll=False)
    lax.fori_loop(0, ROWS_PER_WRK, drain, 0, unroll=False)
    plsc.subcore_barrier()
    pltpu.sync_copy(acc_sh.at[pl.ds(s * NSLICE, NSLICE)],
                    out_hbm.at[c, pl.ds(s * NSLICE, NSLICE)])


def _make_scatter(w, split_cores):
    nrows = ROWS_PER_WRK if split_cores else ROWS_PER_SUB
    scratch = [
        pltpu.VMEM((nrows, LW), jnp.int32),
        pltpu.VMEM((nrows, LW), jnp.int32),
    ]
    scratch += [pltpu.VMEM((LW, w), jnp.float32) for _ in range(NBUF)]
    scratch += [pltpu.VMEM_SHARED((RNP, w), jnp.float32)]
    scratch += [pltpu.SemaphoreType.DMA for _ in range(2 * NBUF)]

    def body(table_hbm, srci_hbm, dsti_hbm, zeros_hbm, out_hbm,
             srci_v, dsti_v, *rest):
        bufs = rest[:NBUF]
        acc_sh = rest[NBUF]
        gsems = rest[NBUF + 1:NBUF + 1 + NBUF]
        ssems = rest[NBUF + 1 + NBUF:]
        c = lax.axis_index("c")
        s = lax.axis_index("s")
        pltpu.sync_copy(zeros_hbm.at[pl.ds(s * NSLICE, NSLICE)],
                        acc_sh.at[pl.ds(s * NSLICE, NSLICE)])
        if split_cores:
            row0 = c * (EROWS // NC) + s * nrows
            pltpu.sync_copy(srci_hbm.at[pl.ds(row0, nrows)], srci_v)
            pltpu.sync_copy(dsti_hbm.at[pl.ds(row0, nrows)], dsti_v)
        else:
            row0 = s * nrows
            pltpu.sync_copy(srci_hbm.at[c, pl.ds(row0, nrows)], srci_v)
            pltpu.sync_copy(dsti_hbm.at[pl.ds(row0, nrows)], dsti_v)
        plsc.subcore_barrier()

        def gather(j, b):
            pltpu.async_copy(table_hbm.at[srci_v.at[j]], bufs[b], gsems[b])

        def gwait(j, b):
            pltpu.make_async_copy(table_hbm.at[srci_v.at[j]], bufs[b],
                                  gsems[b]).wait()

        def scat(j, b):
            pltpu.async_copy(bufs[b], acc_sh.at[dsti_v.at[j]], ssems[b],
                             add=True)

        def swait(j, b):
            pltpu.make_async_copy(bufs[b], acc_sh.at[dsti_v.at[j]],
                                  ssems[b]).wait()

        for b in range(NBUF):
            gather(b, b)

        def step(k, carry):
            j = NBUF * k
            for b in range(NBUF):
                gwait(j + b, b)
                scat(j + b, b)
            for b in range(NBUF):
                swait(j + b, b)

                @pl.when(k < nrows // NBUF - 1)
                def _(b=b, j=j):
                    gather(j + b + NBUF, b)

            return carry

        lax.fori_loop(0, nrows // NBUF, step, 0, unroll=False)
        plsc.subcore_barrier()
        pltpu.sync_copy(acc_sh.at[pl.ds(s * NSLICE, NSLICE)],
                        out_hbm.at[c, pl.ds(s * NSLICE, NSLICE)])

    return pl.kernel(
        body,
        out_type=jax.ShapeDtypeStruct((NC, RNP, w), jnp.float32),
        mesh=_mesh,
        scratch_types=scratch,
        compiler_params=_sc_params,
    )


_scatter_kernel = _make_scatter(32, split_cores=False)
_scatter16_kernel = _make_scatter(16, split_cores=True)


_NB = 1000


def _l0_body(acc_ref, degp_ref, b_ref, h_ref, inv_ref):
    deg = degp_ref[0] + degp_ref[1]
    inv = 1.0 / jnp.maximum(deg, 1.0)
    h = jnp.zeros((_NB, D), jnp.float32)
    for r in range(R):
        h = h + acc_ref[r] * inv[r]
    h_ref[...] = jnp.maximum(h + b_ref[...], 0.0)
    inv_ref[...] = inv


def _l0_dense(acc, degp, b1):
    return pl.pallas_call(
        _l0_body,
        grid=(N // _NB,),
        in_specs=[
            pl.BlockSpec((R, _NB, D), lambda i: (0, i, 0)),
            pl.BlockSpec((NC, R, _NB, 1), lambda i: (0, 0, i, 0)),
            pl.BlockSpec((1, D), lambda i: (0, 0)),
        ],
        out_specs=[
            pl.BlockSpec((_NB, D), lambda i: (i, 0)),
            pl.BlockSpec((R, _NB, 1), lambda i: (0, i, 0)),
        ],
        out_shape=[
            jax.ShapeDtypeStruct((N, D), jnp.float32),
            jax.ShapeDtypeStruct((R, N, 1), jnp.float32),
        ],
    )(acc, degp, b1)


def _mm_body(relu, k, acc_ref, inv_ref, w_ref, b_ref, o_ref):
    y = jnp.zeros((_NB, k), jnp.float32)
    for r in range(R):
        x = acc_ref[r] * inv_ref[r]
        y = y + jnp.dot(x, w_ref[r], preferred_element_type=jnp.float32)
    y = y + b_ref[...]
    o_ref[...] = jnp.maximum(y, 0.0) if relu else y


def _mm_dense(acc, inv, w, b, relu):
    k = w.shape[-1]
    return pl.pallas_call(
        functools.partial(_mm_body, relu, k),
        grid=(N // _NB,),
        in_specs=[
            pl.BlockSpec((R, _NB, D), lambda i: (0, i, 0)),
            pl.BlockSpec((R, _NB, 1), lambda i: (0, i, 0)),
            pl.BlockSpec((R, D, k), lambda i: (0, 0, 0)),
            pl.BlockSpec((1, k), lambda i: (0, 0)),
        ],
        out_specs=pl.BlockSpec((_NB, k), lambda i: (i, 0)),
        out_shape=jax.ShapeDtypeStruct((N, k), jnp.float32),
    )(acc, inv, w, b)


def _y_body(h_ref, w_ref, y_ref):
    for r in range(R):
        y_ref[r] = jnp.dot(h_ref[...], w_ref[r],
                           preferred_element_type=jnp.float32)


def _y_dense(h1, w3):
    return pl.pallas_call(
        _y_body,
        grid=(N // _NB,),
        in_specs=[
            pl.BlockSpec((_NB, D), lambda i: (i, 0)),
            pl.BlockSpec((R, D, 16), lambda i: (0, 0, 0)),
        ],
        out_specs=pl.BlockSpec((R, _NB, 16), lambda i: (0, i, 0)),
        out_shape=jax.ShapeDtypeStruct((R, N, 16), jnp.float32),
    )(h1, w3)


def _out_body(accp_ref, inv_ref, b_ref, o_ref):
    y = jnp.zeros((_NB, 16), jnp.float32)
    for r in range(R):
        y = y + (accp_ref[0, r] + accp_ref[1, r]) * inv_ref[r]
    o_ref[...] = y + b_ref[...]


def _out_dense(accp, inv, b3):
    return pl.pallas_call(
        _out_body,
        grid=(N // _NB,),
        in_specs=[
            pl.BlockSpec((NC, R, _NB, 16), lambda i: (0, 0, i, 0)),
            pl.BlockSpec((R, _NB, 1), lambda i: (0, i, 0)),
            pl.BlockSpec((1, 16), lambda i: (0, 0)),
        ],
        out_specs=pl.BlockSpec((_NB, 16), lambda i: (i, 0)),
        out_shape=jax.ShapeDtypeStruct((N, 16), jnp.float32),
    )(accp, inv, b3)


def _aggregate(x, srci, dsti, zeros32):
    chunks = []
    for p in range(2):
        table = jnp.concatenate(
            [x[:, 64 * p:64 * p + 32], x[:, 64 * p + 32:64 * p + 64]], axis=0)
        chunks.append(_scatter_kernel(table, srci, dsti, zeros32))
    acc = jnp.concatenate(
        [chunks[0][0, :RN], chunks[0][1, :RN],
         chunks[1][0, :RN], chunks[1][1, :RN]], axis=-1)
    return acc.reshape(R, N, D)


def kernel(edge_src, edge_dst, edge_type, embed, h_bias1, W2, b2, W3, b3):
    pad = EPAD - E
    src_p = jnp.concatenate([edge_src, jnp.zeros((pad,), jnp.int32)])
    srci = jnp.stack([src_p, src_p + N]).reshape(NC, EROWS, LW)
    typ_p = jnp.concatenate([edge_type, jnp.zeros((pad,), jnp.int32)])
    srci2 = (typ_p * N + src_p).reshape(EROWS, LW)
    dst_p = jnp.concatenate(
        [edge_type * N + edge_dst, jnp.full((pad,), RN, jnp.int32)])
    dsti = dst_p.reshape(EROWS, LW)

    zeros32 = jnp.zeros((RNP, 32), jnp.float32)
    zeros8 = jnp.zeros((RNP, 8), jnp.float32)
    ones8 = jnp.ones((LW, 8), jnp.float32)

    degp = _deg_kernel(dsti, ones8, zeros8)
    degp = degp[:, :RN, 0].reshape(NC, R, N, 1)

    acc0 = _aggregate(embed, srci, dsti, zeros32)
    h0, inv = _l0_dense(acc0, degp, h_bias1.reshape(1, D))

    acc1 = _aggregate(h0, srci, dsti, zeros32)
    h1 = _mm_dense(acc1, inv, W2, b2.reshape(1, D), relu=True)

    y = _y_dense(h1, W3).reshape(RN, 16)
    zeros16 = jnp.zeros((RNP, 16), jnp.float32)
    accp = _scatter16_kernel(y, srci2, dsti, zeros16)
    accp = accp[:, :RN].reshape(NC, R, N, 16)
    out = _out_dense(accp, inv, b3.reshape(1, 16))
    return out


# speedup vs baseline: 1.0992x; 1.0992x over previous
import functools

import jax
import jax.numpy as jnp
from jax import lax
from jax.experimental import pallas as pl
from jax.experimental.pallas import tpu as pltpu, tpu_sc as plsc

N = 10000
E = 320000
R = 4
RN = R * N
D = 128
NC = 2
NS = 16
LW = 128
EPAD = 327680
EROWS = EPAD // LW
ROWS_PER_SUB = EROWS // NS
ROWS_PER_WRK = EROWS // (NC * NS)
NPAD = 10240
RNP = R * NPAD
NSLICE = RNP // NS
NBUF = 2

_mesh = plsc.VectorSubcoreMesh(core_axis_name="c", subcore_axis_name="s")
_sc_params = pltpu.CompilerParams(use_tc_tiling_on_sc=False)


@functools.partial(
    pl.kernel,
    out_type=jax.ShapeDtypeStruct((NC, RNP, 8), jnp.float32),
    mesh=_mesh,
    scratch_types=[
        pltpu.VMEM((ROWS_PER_WRK, LW), jnp.int32),
        pltpu.VMEM((LW, 8), jnp.float32),
        pltpu.VMEM_SHARED((RNP, 8), jnp.float32),
        pltpu.SemaphoreType.DMA,
    ],
    compiler_params=_sc_params,
)
def _deg_kernel(dsti_hbm, ones_hbm, zeros_hbm, out_hbm, dsti_v, ones_v, acc_sh,
                sem):
    c = lax.axis_index("c")
    s = lax.axis_index("s")
    wid = s * NC + c
    pltpu.sync_copy(zeros_hbm.at[pl.ds(s * NSLICE, NSLICE)],
                    acc_sh.at[pl.ds(s * NSLICE, NSLICE)])
    pltpu.sync_copy(dsti_hbm.at[pl.ds(wid * ROWS_PER_WRK, ROWS_PER_WRK)], dsti_v)
    pltpu.sync_copy(ones_hbm, ones_v)
    plsc.subcore_barrier()

    def fire(j, carry):
        pltpu.async_copy(ones_v, acc_sh.at[dsti_v.at[j]], sem, add=True)
        return carry

    def drain(j, carry):
        pltpu.make_async_copy(ones_v, acc_sh.at[dsti_v.at[j]], sem).wait()
        return carry

    lax.fori_loop(0, ROWS_PER_WRK, fire, 0, unroll=False)
    lax.fori_loop(0, ROWS_PER_WRK, drain, 0, unroll=False)
    plsc.subcore_barrier()
    pltpu.sync_copy(acc_sh.at[pl.ds(s * NSLICE, NSLICE)],
                    out_hbm.at[c, pl.ds(s * NSLICE, NSLICE)])


def _make_scatter(w, split_cores):
    nrows = ROWS_PER_WRK if split_cores else ROWS_PER_SUB
    scratch = [
        pltpu.VMEM((nrows, LW), jnp.int32),
        pltpu.VMEM((nrows, LW), jnp.int32),
    ]
    scratch += [pltpu.VMEM((LW, w), jnp.float32) for _ in range(NBUF)]
    scratch += [pltpu.VMEM_SHARED((RNP, w), jnp.float32)]
    scratch += [pltpu.SemaphoreType.DMA for _ in range(2 * NBUF)]

    def body(table_hbm, srci_hbm, dsti_hbm, zeros_hbm, out_hbm,
             srci_v, dsti_v, *rest):
        bufs = rest[:NBUF]
        acc_sh = rest[NBUF]
        gsems = rest[NBUF + 1:NBUF + 1 + NBUF]
        ssems = rest[NBUF + 1 + NBUF:]
        c = lax.axis_index("c")
        s = lax.axis_index("s")
        pltpu.sync_copy(zeros_hbm.at[pl.ds(s * NSLICE, NSLICE)],
                        acc_sh.at[pl.ds(s * NSLICE, NSLICE)])
        if split_cores:
            row0 = c * (EROWS // NC) + s * nrows
            pltpu.sync_copy(srci_hbm.at[pl.ds(row0, nrows)], srci_v)
            pltpu.sync_copy(dsti_hbm.at[pl.ds(row0, nrows)], dsti_v)
        else:
            row0 = s * nrows
            pltpu.sync_copy(srci_hbm.at[c, pl.ds(row0, nrows)], srci_v)
            pltpu.sync_copy(dsti_hbm.at[pl.ds(row0, nrows)], dsti_v)
        plsc.subcore_barrier()

        def gather(j, b):
            pltpu.async_copy(table_hbm.at[srci_v.at[j]], bufs[b], gsems[b])

        def gwait(j, b):
            pltpu.make_async_copy(table_hbm.at[srci_v.at[j]], bufs[b],
                                  gsems[b]).wait()

        def scat(j, b):
            pltpu.async_copy(bufs[b], acc_sh.at[dsti_v.at[j]], ssems[b],
                             add=True)

        def swait(j, b):
            pltpu.make_async_copy(bufs[b], acc_sh.at[dsti_v.at[j]],
                                  ssems[b]).wait()

        for b in range(NBUF):
            gather(b, b)

        def step(k, carry):
            j = NBUF * k
            for b in range(NBUF):
                gwait(j + b, b)
                scat(j + b, b)
            for b in range(NBUF):
                swait(j + b, b)

                @pl.when(k < nrows // NBUF - 1)
                def _(b=b, j=j):
                    gather(j + b + NBUF, b)

            return carry

        lax.fori_loop(0, nrows // NBUF, step, 0, unroll=False)
        plsc.subcore_barrier()
        pltpu.sync_copy(acc_sh.at[pl.ds(s * NSLICE, NSLICE)],
                        out_hbm.at[c, pl.ds(s * NSLICE, NSLICE)])

    return pl.kernel(
        body,
        out_type=jax.ShapeDtypeStruct((NC, RNP, w), jnp.float32),
        mesh=_mesh,
        scratch_types=scratch,
        compiler_params=_sc_params,
    )


_scatter_kernel = _make_scatter(32, split_cores=False)
_scatter16_kernel = _make_scatter(16, split_cores=True)


_NB = 1000


def _chunk_cols(pa_ref, pb_ref, r):
    return [pa_ref[0, r], pa_ref[1, r], pb_ref[0, r], pb_ref[1, r]]


def _l0_body(pa_ref, pb_ref, degp_ref, b_ref, ta_ref, tb_ref, inv_ref):
    deg = degp_ref[0, :, :, 0:1] + degp_ref[1, :, :, 0:1]
    inv = 1.0 / jnp.maximum(deg, 1.0)
    outs = (ta_ref, ta_ref, tb_ref, tb_ref)
    for k in range(4):
        h = jnp.zeros((_NB, 32), jnp.float32)
        for r in range(R):
            h = h + _chunk_cols(pa_ref, pb_ref, r)[k] * inv[r]
        outs[k][k % 2] = jnp.maximum(h + b_ref[:, 32 * k:32 * k + 32], 0.0)
    inv_ref[...] = inv


def _l0_dense(pa, pb, degp, b1):
    acc_spec = pl.BlockSpec((NC, R, _NB, 32), lambda i: (0, 0, i, 0))
    tab_spec = pl.BlockSpec((2, _NB, 32), lambda i: (0, i, 0))
    return pl.pallas_call(
        _l0_body,
        grid=(N // _NB,),
        in_specs=[
            acc_spec,
            acc_spec,
            pl.BlockSpec((NC, R, _NB, 8), lambda i: (0, 0, i, 0)),
            pl.BlockSpec((1, D), lambda i: (0, 0)),
        ],
        out_specs=[
            tab_spec,
            tab_spec,
            pl.BlockSpec((R, _NB, 1), lambda i: (0, i, 0)),
        ],
        out_shape=[
            jax.ShapeDtypeStruct((2, N, 32), jnp.float32),
            jax.ShapeDtypeStruct((2, N, 32), jnp.float32),
            jax.ShapeDtypeStruct((R, N, 1), jnp.float32),
        ],
    )(pa, pb, degp, b1)


def _l1_body(pa_ref, pb_ref, inv_ref, w2_ref, b2_ref, w3_ref, y_ref):
    h = jnp.zeros((_NB, D), jnp.float32)
    for r in range(R):
        x = jnp.concatenate(_chunk_cols(pa_ref, pb_ref, r), axis=-1)
        x = x * inv_ref[r]
        h = h + jnp.dot(x, w2_ref[r], preferred_element_type=jnp.float32)
    h = jnp.maximum(h + b2_ref[...], 0.0)
    for r in range(R):
        y_ref[r] = jnp.dot(h, w3_ref[r], preferred_element_type=jnp.float32)


def _l1_dense(pa, pb, inv, w2, b2, w3):
    acc_spec = pl.BlockSpec((NC, R, _NB, 32), lambda i: (0, 0, i, 0))
    return pl.pallas_call(
        _l1_body,
        grid=(N // _NB,),
        in_specs=[
            acc_spec,
            acc_spec,
            pl.BlockSpec((R, _NB, 1), lambda i: (0, i, 0)),
            pl.BlockSpec((R, D, D), lambda i: (0, 0, 0)),
            pl.BlockSpec((1, D), lambda i: (0, 0)),
            pl.BlockSpec((R, D, 16), lambda i: (0, 0, 0)),
        ],
        out_specs=pl.BlockSpec((R, _NB, 16), lambda i: (0, i, 0)),
        out_shape=jax.ShapeDtypeStruct((R, N, 16), jnp.float32),
    )(pa, pb, inv, w2, b2, w3)


def _out_body(accp_ref, inv_ref, b_ref, o_ref):
    y = jnp.zeros((_NB, 16), jnp.float32)
    for r in range(R):
        y = y + (accp_ref[0, r] + accp_ref[1, r]) * inv_ref[r]
    o_ref[...] = y + b_ref[...]


def _out_dense(accp, inv, b3):
    return pl.pallas_call(
        _out_body,
        grid=(N // _NB,),
        in_specs=[
            pl.BlockSpec((NC, R, _NB, 16), lambda i: (0, 0, i, 0)),
            pl.BlockSpec((R, _NB, 1), lambda i: (0, i, 0)),
            pl.BlockSpec((1, 16), lambda i: (0, 0)),
        ],
        out_specs=pl.BlockSpec((_NB, 16), lambda i: (i, 0)),
        out_shape=jax.ShapeDtypeStruct((N, 16), jnp.float32),
    )(accp, inv, b3)


def _aggregate(tabs, srci, dsti, zeros32):
    out = []
    for p in range(2):
        acc = _scatter_kernel(tabs[p].reshape(2 * N, 32), srci, dsti, zeros32)
        out.append(acc.reshape(NC, R, NPAD, 32))
    return out


def kernel(edge_src, edge_dst, edge_type, embed, h_bias1, W2, b2, W3, b3):
    pad = EPAD - E
    src_p = jnp.concatenate([edge_src, jnp.zeros((pad,), jnp.int32)])
    srci = jnp.stack([src_p, src_p + N]).reshape(NC, EROWS, LW)
    typ_p = jnp.concatenate([edge_type, jnp.zeros((pad,), jnp.int32)])
    srci2 = (typ_p * N + src_p).reshape(EROWS, LW)
    dst_p = jnp.concatenate(
        [edge_type * NPAD + edge_dst, jnp.full((pad,), N, jnp.int32)])
    dsti = dst_p.reshape(EROWS, LW)

    zeros32 = jnp.zeros((RNP, 32), jnp.float32)
    zeros8 = jnp.zeros((RNP, 8), jnp.float32)
    ones8 = jnp.ones((LW, 8), jnp.float32)

    degp = _deg_kernel(dsti, ones8, zeros8)
    degp = degp.reshape(NC, R, NPAD, 8)

    emb_tabs = [
        jnp.stack([embed[:, 64 * p:64 * p + 32],
                   embed[:, 64 * p + 32:64 * p + 64]]) for p in range(2)]
    acc0 = _aggregate(emb_tabs, srci, dsti, zeros32)
    ta, tb, inv = _l0_dense(acc0[0], acc0[1], degp, h_bias1.reshape(1, D))

    acc1 = _aggregate([ta, tb], srci, dsti, zeros32)
    y = _l1_dense(acc1[0], acc1[1], inv, W2, b2.reshape(1, D), W3)

    zeros16 = jnp.zeros((RNP, 16), jnp.float32)
    accp = _scatter16_kernel(y.reshape(RN, 16), srci2, dsti, zeros16)
    accp = accp.reshape(NC, R, NPAD, 16)
    out = _out_dense(accp, inv, b3.reshape(1, 16))
    return out


# speedup vs baseline: 1.2090x; 1.0999x over previous
import functools

import jax
import jax.numpy as jnp
from jax import lax
from jax.experimental import pallas as pl
from jax.experimental.pallas import tpu as pltpu, tpu_sc as plsc

N = 10000
E = 320000
R = 4
RN = R * N
D = 128
NC = 2
NS = 16
LW = 128
EPAD = 327680
EROWS = EPAD // LW
ROWS_PER_SUB = EROWS // NS
ROWS_PER_WRK = EROWS // (NC * NS)
NPAD = 10240
RNP = R * NPAD
NSLICE = RNP // NS
NBUF = 2

_mesh = plsc.VectorSubcoreMesh(core_axis_name="c", subcore_axis_name="s")
_sc_params = pltpu.CompilerParams(use_tc_tiling_on_sc=False)


@functools.partial(
    pl.kernel,
    out_type=jax.ShapeDtypeStruct((NC, RNP, 8), jnp.float32),
    mesh=_mesh,
    scratch_types=[
        pltpu.VMEM((ROWS_PER_WRK, LW), jnp.int32),
        pltpu.VMEM((LW, 8), jnp.float32),
        pltpu.VMEM_SHARED((RNP, 8), jnp.float32),
        pltpu.SemaphoreType.DMA,
    ],
    compiler_params=_sc_params,
)
def _deg_kernel(dsti_hbm, ones_hbm, zeros_hbm, out_hbm, dsti_v, ones_v, acc_sh,
                sem):
    c = lax.axis_index("c")
    s = lax.axis_index("s")
    wid = s * NC + c
    pltpu.sync_copy(zeros_hbm.at[pl.ds(s * NSLICE, NSLICE)],
                    acc_sh.at[pl.ds(s * NSLICE, NSLICE)])
    pltpu.sync_copy(dsti_hbm.at[pl.ds(wid * ROWS_PER_WRK, ROWS_PER_WRK)], dsti_v)
    pltpu.sync_copy(ones_hbm, ones_v)
    plsc.subcore_barrier()

    def fire(j, carry):
        pltpu.async_copy(ones_v, acc_sh.at[dsti_v.at[j]], sem, add=True)
        return carry

    def drain(j, carry):
        pltpu.make_async_copy(ones_v, acc_sh.at[dsti_v.at[j]], sem).wait()
        return carry

    lax.fori_loop(0, ROWS_PER_WRK, fire, 0, unroll=False)
    lax.fori_loop(0, ROWS_PER_WRK, drain, 0, unroll=False)
    plsc.subcore_barrier()
    pltpu.sync_copy(acc_sh.at[pl.ds(s * NSLICE, NSLICE)],
                    out_hbm.at[c, pl.ds(s * NSLICE, NSLICE)])


def _make_scatter(w, split_cores):
    nrows = ROWS_PER_WRK if split_cores else ROWS_PER_SUB
    scratch = [
        pltpu.VMEM((nrows, LW), jnp.int32),
        pltpu.VMEM((nrows, LW), jnp.int32),
    ]
    scratch += [pltpu.VMEM((LW, w), jnp.float32) for _ in range(NBUF)]
    scratch += [pltpu.VMEM_SHARED((RNP, w), jnp.float32)]
    scratch += [pltpu.SemaphoreType.DMA for _ in range(2 * NBUF)]

    def body(table_hbm, srci_hbm, dsti_hbm, zeros_hbm, out_hbm,
             srci_v, dsti_v, *rest):
        bufs = rest[:NBUF]
        acc_sh = rest[NBUF]
        gsems = rest[NBUF + 1:NBUF + 1 + NBUF]
        ssems = rest[NBUF + 1 + NBUF:]
        c = lax.axis_index("c")
        s = lax.axis_index("s")
        pltpu.sync_copy(zeros_hbm.at[pl.ds(s * NSLICE, NSLICE)],
                        acc_sh.at[pl.ds(s * NSLICE, NSLICE)])
        if split_cores:
            row0 = c * (EROWS // NC) + s * nrows
            pltpu.sync_copy(srci_hbm.at[pl.ds(row0, nrows)], srci_v)
            pltpu.sync_copy(dsti_hbm.at[pl.ds(row0, nrows)], dsti_v)
        else:
            row0 = s * nrows
            pltpu.sync_copy(srci_hbm.at[c, pl.ds(row0, nrows)], srci_v)
            pltpu.sync_copy(dsti_hbm.at[pl.ds(row0, nrows)], dsti_v)
        plsc.subcore_barrier()

        def gather(j, b):
            pltpu.async_copy(table_hbm.at[srci_v.at[j]], bufs[b], gsems[b])

        def gwait(j, b):
            pltpu.make_async_copy(table_hbm.at[srci_v.at[j]], bufs[b],
                                  gsems[b]).wait()

        def scat(j, b):
            pltpu.async_copy(bufs[b], acc_sh.at[dsti_v.at[j]], ssems[b],
                             add=True)

        def swait(j, b):
            pltpu.make_async_copy(bufs[b], acc_sh.at[dsti_v.at[j]],
                                  ssems[b]).wait()

        for b in range(NBUF):
            gather(b, b)

        def step(k, carry):
            j = NBUF * k
            for b in range(NBUF):
                gwait(j + b, b)
                scat(j + b, b)
            for b in range(NBUF):
                swait(j + b, b)

                @pl.when(k < nrows // NBUF - 1)
                def _(b=b, j=j):
                    gather(j + b + NBUF, b)

            return carry

        lax.fori_loop(0, nrows // NBUF, step, 0, unroll=False)
        plsc.subcore_barrier()
        pltpu.sync_copy(acc_sh.at[pl.ds(s * NSLICE, NSLICE)],
                        out_hbm.at[c, pl.ds(s * NSLICE, NSLICE)])

    return pl.kernel(
        body,
        out_type=jax.ShapeDtypeStruct((NC, RNP, w), jnp.float32),
        mesh=_mesh,
        scratch_types=scratch,
        compiler_params=_sc_params,
    )


_scatter_kernel = _make_scatter(32, split_cores=False)
_scatter16_kernel = _make_scatter(16, split_cores=True)


_NB = 1000


def _chunk_cols(pa_ref, pb_ref, r):
    return [pa_ref[0, r], pa_ref[1, r], pb_ref[0, r], pb_ref[1, r]]


def _half_body(emit_inv, ph_ref, degp_ref, b_ref, *outs):
    t_ref = outs[0]
    deg = degp_ref[0, :, :, 0:1] + degp_ref[1, :, :, 0:1]
    inv = 1.0 / jnp.maximum(deg, 1.0)
    for k in range(2):
        h = jnp.zeros((_NB, 32), jnp.float32)
        for r in range(R):
            h = h + ph_ref[k, r] * inv[r]
        t_ref[k] = jnp.maximum(h + b_ref[:, 32 * k:32 * k + 32], 0.0)
    if emit_inv:
        outs[1][...] = inv


def _l0_half(ph, degp, bh, emit_inv):
    out_specs = [pl.BlockSpec((2, _NB, 32), lambda i: (0, i, 0))]
    out_shape = [jax.ShapeDtypeStruct((2, N, 32), jnp.float32)]
    if emit_inv:
        out_specs.append(pl.BlockSpec((R, _NB, 1), lambda i: (0, i, 0)))
        out_shape.append(jax.ShapeDtypeStruct((R, N, 1), jnp.float32))
    return pl.pallas_call(
        functools.partial(_half_body, emit_inv),
        grid=(N // _NB,),
        in_specs=[
            pl.BlockSpec((NC, R, _NB, 32), lambda i: (0, 0, i, 0)),
            pl.BlockSpec((NC, R, _NB, 8), lambda i: (0, 0, i, 0)),
            pl.BlockSpec((1, 64), lambda i: (0, 0)),
        ],
        out_specs=out_specs,
        out_shape=out_shape,
    )(ph, degp, bh)


def _l1a_body(pa_ref, inv_ref, w2_ref, p_ref):
    p = jnp.zeros((_NB, D), jnp.float32)
    for r in range(R):
        x = jnp.concatenate([pa_ref[0, r], pa_ref[1, r]], axis=-1)
        x = x * inv_ref[r]
        p = p + jnp.dot(x, w2_ref[r][0:64], preferred_element_type=jnp.float32)
    p_ref[...] = p


def _l1a_dense(pa, inv, w2):
    return pl.pallas_call(
        _l1a_body,
        grid=(N // _NB,),
        in_specs=[
            pl.BlockSpec((NC, R, _NB, 32), lambda i: (0, 0, i, 0)),
            pl.BlockSpec((R, _NB, 1), lambda i: (0, i, 0)),
            pl.BlockSpec((R, D, D), lambda i: (0, 0, 0)),
        ],
        out_specs=pl.BlockSpec((_NB, D), lambda i: (i, 0)),
        out_shape=jax.ShapeDtypeStruct((N, D), jnp.float32),
    )(pa, inv, w2)


def _l1b_body(pb_ref, inv_ref, w2_ref, b2_ref, w3_ref, p_ref, y_ref):
    h = p_ref[...]
    for r in range(R):
        x = jnp.concatenate([pb_ref[0, r], pb_ref[1, r]], axis=-1)
        x = x * inv_ref[r]
        h = h + jnp.dot(x, w2_ref[r][64:D], preferred_element_type=jnp.float32)
    h = jnp.maximum(h + b2_ref[...], 0.0)
    for r in range(R):
        y_ref[r] = jnp.dot(h, w3_ref[r], preferred_element_type=jnp.float32)


def _l1b_dense(pb, inv, w2, b2, w3, partial):
    return pl.pallas_call(
        _l1b_body,
        grid=(N // _NB,),
        in_specs=[
            pl.BlockSpec((NC, R, _NB, 32), lambda i: (0, 0, i, 0)),
            pl.BlockSpec((R, _NB, 1), lambda i: (0, i, 0)),
            pl.BlockSpec((R, D, D), lambda i: (0, 0, 0)),
            pl.BlockSpec((1, D), lambda i: (0, 0)),
            pl.BlockSpec((R, D, 16), lambda i: (0, 0, 0)),
            pl.BlockSpec((_NB, D), lambda i: (i, 0)),
        ],
        out_specs=pl.BlockSpec((R, _NB, 16), lambda i: (0, i, 0)),
        out_shape=jax.ShapeDtypeStruct((R, N, 16), jnp.float32),
    )(pb, inv, w2, b2, w3, partial)


def _out_body(accp_ref, inv_ref, b_ref, o_ref):
    y = jnp.zeros((_NB, 16), jnp.float32)
    for r in range(R):
        y = y + (accp_ref[0, r] + accp_ref[1, r]) * inv_ref[r]
    o_ref[...] = y + b_ref[...]


def _out_dense(accp, inv, b3):
    return pl.pallas_call(
        _out_body,
        grid=(N // _NB,),
        in_specs=[
            pl.BlockSpec((NC, R, _NB, 16), lambda i: (0, 0, i, 0)),
            pl.BlockSpec((R, _NB, 1), lambda i: (0, i, 0)),
            pl.BlockSpec((1, 16), lambda i: (0, 0)),
        ],
        out_specs=pl.BlockSpec((_NB, 16), lambda i: (i, 0)),
        out_shape=jax.ShapeDtypeStruct((N, 16), jnp.float32),
    )(accp, inv, b3)


def _aggregate(tabs, srci, dsti, zeros32):
    out = []
    for p in range(2):
        acc = _scatter_kernel(tabs[p].reshape(2 * N, 32), srci, dsti, zeros32)
        out.append(acc.reshape(NC, R, NPAD, 32))
    return out


def kernel(edge_src, edge_dst, edge_type, embed, h_bias1, W2, b2, W3, b3):
    pad = EPAD - E
    src_p = jnp.concatenate([edge_src, jnp.zeros((pad,), jnp.int32)])
    srci = jnp.stack([src_p, src_p + N]).reshape(NC, EROWS, LW)
    typ_p = jnp.concatenate([edge_type, jnp.zeros((pad,), jnp.int32)])
    srci2 = (typ_p * N + src_p).reshape(EROWS, LW)
    dst_p = jnp.concatenate(
        [edge_type * NPAD + edge_dst, jnp.full((pad,), N, jnp.int32)])
    dsti = dst_p.reshape(EROWS, LW)

    zeros32 = jnp.zeros((RNP, 32), jnp.float32)
    zeros8 = jnp.zeros((RNP, 8), jnp.float32)
    ones8 = jnp.ones((LW, 8), jnp.float32)

    degp = _deg_kernel(dsti, ones8, zeros8)
    degp = degp.reshape(NC, R, NPAD, 8)

    emb_tabs = [
        jnp.stack([embed[:, 64 * p:64 * p + 32],
                   embed[:, 64 * p + 32:64 * p + 64]]) for p in range(2)]
    accA = _scatter_kernel(emb_tabs[0].reshape(2 * N, 32), srci, dsti,
                           zeros32).reshape(NC, R, NPAD, 32)
    accB = _scatter_kernel(emb_tabs[1].reshape(2 * N, 32), srci, dsti,
                           zeros32).reshape(NC, R, NPAD, 32)
    b1 = h_bias1.reshape(1, D)
    ta = _l0_half(accA, degp, b1[:, 0:64], emit_inv=False)[0]
    tb, inv = _l0_half(accB, degp, b1[:, 64:D], emit_inv=True)

    acc1A = _scatter_kernel(ta.reshape(2 * N, 32), srci, dsti,
                            zeros32).reshape(NC, R, NPAD, 32)
    acc1B = _scatter_kernel(tb.reshape(2 * N, 32), srci, dsti,
                            zeros32).reshape(NC, R, NPAD, 32)
    partial = _l1a_dense(acc1A, inv, W2)
    y = _l1b_dense(acc1B, inv, W2, b2.reshape(1, D), W3, partial)

    zeros16 = jnp.zeros((RNP, 16), jnp.float32)
    accp = _scatter16_kernel(y.reshape(RN, 16), srci2, dsti, zeros16)
    accp = accp.reshape(NC, R, NPAD, 16)
    out = _out_dense(accp, inv, b3.reshape(1, 16))
    return out


# speedup vs baseline: 1.2307x; 1.0179x over previous
import functools

import jax
import jax.numpy as jnp
from jax import lax
from jax.experimental import pallas as pl
from jax.experimental.pallas import tpu as pltpu, tpu_sc as plsc

N = 10000
E = 320000
R = 4
RN = R * N
D = 128
NC = 2
NS = 16
LW = 128
EPAD = 327680
EROWS = EPAD // LW
ROWS_PER_SUB = EROWS // NS
ROWS_PER_WRK = EROWS // (NC * NS)
NPAD = 10240
RNP = R * NPAD
NSLICE = RNP // NS
NBUF = 2

_mesh = plsc.VectorSubcoreMesh(core_axis_name="c", subcore_axis_name="s")
_sc_params = pltpu.CompilerParams(use_tc_tiling_on_sc=False)


@functools.partial(
    pl.kernel,
    out_type=jax.ShapeDtypeStruct((NC, RNP, 8), jnp.float32),
    mesh=_mesh,
    scratch_types=[
        pltpu.VMEM((ROWS_PER_WRK, LW), jnp.int32),
        pltpu.VMEM((LW, 8), jnp.float32),
        pltpu.VMEM_SHARED((RNP, 8), jnp.float32),
        pltpu.SemaphoreType.DMA,
    ],
    compiler_params=_sc_params,
)
def _deg_kernel(dsti_hbm, ones_hbm, zeros_hbm, out_hbm, dsti_v, ones_v, acc_sh,
                sem):
    c = lax.axis_index("c")
    s = lax.axis_index("s")
    wid = s * NC + c
    pltpu.sync_copy(zeros_hbm.at[pl.ds(s * NSLICE, NSLICE)],
                    acc_sh.at[pl.ds(s * NSLICE, NSLICE)])
    pltpu.sync_copy(dsti_hbm.at[pl.ds(wid * ROWS_PER_WRK, ROWS_PER_WRK)], dsti_v)
    pltpu.sync_copy(ones_hbm, ones_v)
    plsc.subcore_barrier()

    def fire(j, carry):
        pltpu.async_copy(ones_v, acc_sh.at[dsti_v.at[j]], sem, add=True)
        return carry

    def drain(j, carry):
        pltpu.make_async_copy(ones_v, acc_sh.at[dsti_v.at[j]], sem).wait()
        return carry

    lax.fori_loop(0, ROWS_PER_WRK, fire, 0, unroll=False)
    lax.fori_loop(0, ROWS_PER_WRK, drain, 0, unroll=False)
    plsc.subcore_barrier()
    pltpu.sync_copy(acc_sh.at[pl.ds(s * NSLICE, NSLICE)],
                    out_hbm.at[c, pl.ds(s * NSLICE, NSLICE)])


def _make_scatter(w, split_cores, nbuf=NBUF):
    nrows = ROWS_PER_WRK if split_cores else ROWS_PER_SUB
    scratch = [
        pltpu.VMEM((nrows, LW), jnp.int32),
        pltpu.VMEM((nrows, LW), jnp.int32),
    ]
    scratch += [pltpu.VMEM((LW, w), jnp.float32) for _ in range(nbuf)]
    scratch += [pltpu.VMEM_SHARED((RNP, w), jnp.float32)]
    scratch += [pltpu.SemaphoreType.DMA for _ in range(2 * nbuf)]

    def body(table_hbm, srci_hbm, dsti_hbm, zeros_hbm, out_hbm,
             srci_v, dsti_v, *rest):
        bufs = rest[:nbuf]
        acc_sh = rest[nbuf]
        gsems = rest[nbuf + 1:nbuf + 1 + nbuf]
        ssems = rest[nbuf + 1 + nbuf:]
        c = lax.axis_index("c")
        s = lax.axis_index("s")
        zdesc = pltpu.async_copy(zeros_hbm.at[pl.ds(s * NSLICE, NSLICE)],
                                 acc_sh.at[pl.ds(s * NSLICE, NSLICE)],
                                 gsems[0])
        if split_cores:
            row0 = c * (EROWS // NC) + s * nrows
            sdesc = pltpu.async_copy(srci_hbm.at[pl.ds(row0, nrows)], srci_v,
                                     gsems[1])
            ddesc = pltpu.async_copy(dsti_hbm.at[pl.ds(row0, nrows)], dsti_v,
                                     ssems[0])
        else:
            row0 = s * nrows
            sdesc = pltpu.async_copy(srci_hbm.at[c, pl.ds(row0, nrows)],
                                     srci_v, gsems[1])
            ddesc = pltpu.async_copy(dsti_hbm.at[pl.ds(row0, nrows)], dsti_v,
                                     ssems[0])
        zdesc.wait()
        sdesc.wait()
        ddesc.wait()
        plsc.subcore_barrier()

        def gather(j, b):
            pltpu.async_copy(table_hbm.at[srci_v.at[j]], bufs[b], gsems[b])

        def gwait(j, b):
            pltpu.make_async_copy(table_hbm.at[srci_v.at[j]], bufs[b],
                                  gsems[b]).wait()

        def scat(j, b):
            pltpu.async_copy(bufs[b], acc_sh.at[dsti_v.at[j]], ssems[b],
                             add=True)

        def swait(j, b):
            pltpu.make_async_copy(bufs[b], acc_sh.at[dsti_v.at[j]],
                                  ssems[b]).wait()

        for b in range(nbuf):
            gather(b, b)

        def step(k, carry):
            j = nbuf * k
            for b in range(nbuf):
                gwait(j + b, b)
                scat(j + b, b)
            for b in range(nbuf):
                swait(j + b, b)

                @pl.when(k < nrows // nbuf - 1)
                def _(b=b, j=j):
                    gather(j + b + nbuf, b)

            return carry

        lax.fori_loop(0, nrows // nbuf, step, 0, unroll=False)
        plsc.subcore_barrier()
        pltpu.sync_copy(acc_sh.at[pl.ds(s * NSLICE, NSLICE)],
                        out_hbm.at[c, pl.ds(s * NSLICE, NSLICE)])

    return pl.kernel(
        body,
        out_type=jax.ShapeDtypeStruct((NC, RNP, w), jnp.float32),
        mesh=_mesh,
        scratch_types=scratch,
        compiler_params=_sc_params,
    )


_scatter_kernel = _make_scatter(32, split_cores=False)
_scatter16_kernel = _make_scatter(16, split_cores=True, nbuf=8)


_NB = 1000


def _chunk_cols(pa_ref, pb_ref, r):
    return [pa_ref[0, r], pa_ref[1, r], pb_ref[0, r], pb_ref[1, r]]


def _half_body(emit_inv, ph_ref, degp_ref, b_ref, *outs):
    t_ref = outs[0]
    deg = degp_ref[0, :, :, 0:1] + degp_ref[1, :, :, 0:1]
    inv = 1.0 / jnp.maximum(deg, 1.0)
    for k in range(2):
        h = jnp.zeros((_NB, 32), jnp.float32)
        for r in range(R):
            h = h + ph_ref[k, r] * inv[r]
        t_ref[k] = jnp.maximum(h + b_ref[:, 32 * k:32 * k + 32], 0.0)
    if emit_inv:
        outs[1][...] = inv


def _l0_half(ph, degp, bh, emit_inv):
    out_specs = [pl.BlockSpec((2, _NB, 32), lambda i: (0, i, 0))]
    out_shape = [jax.ShapeDtypeStruct((2, N, 32), jnp.float32)]
    if emit_inv:
        out_specs.append(pl.BlockSpec((R, _NB, 1), lambda i: (0, i, 0)))
        out_shape.append(jax.ShapeDtypeStruct((R, N, 1), jnp.float32))
    return pl.pallas_call(
        functools.partial(_half_body, emit_inv),
        grid=(N // _NB,),
        in_specs=[
            pl.BlockSpec((NC, R, _NB, 32), lambda i: (0, 0, i, 0)),
            pl.BlockSpec((NC, R, _NB, 8), lambda i: (0, 0, i, 0)),
            pl.BlockSpec((1, 64), lambda i: (0, 0)),
        ],
        out_specs=out_specs,
        out_shape=out_shape,
    )(ph, degp, bh)


def _l1a_body(pa_ref, inv_ref, w2_ref, p_ref):
    p = jnp.zeros((_NB, D), jnp.float32)
    for r in range(R):
        x = jnp.concatenate([pa_ref[0, r], pa_ref[1, r]], axis=-1)
        x = x * inv_ref[r]
        p = p + jnp.dot(x, w2_ref[r][0:64], preferred_element_type=jnp.float32)
    p_ref[...] = p


def _l1a_dense(pa, inv, w2):
    return pl.pallas_call(
        _l1a_body,
        grid=(N // _NB,),
        in_specs=[
            pl.BlockSpec((NC, R, _NB, 32), lambda i: (0, 0, i, 0)),
            pl.BlockSpec((R, _NB, 1), lambda i: (0, i, 0)),
            pl.BlockSpec((R, D, D), lambda i: (0, 0, 0)),
        ],
        out_specs=pl.BlockSpec((_NB, D), lambda i: (i, 0)),
        out_shape=jax.ShapeDtypeStruct((N, D), jnp.float32),
    )(pa, inv, w2)


def _l1b_body(pb_ref, inv_ref, w2_ref, b2_ref, w3_ref, p_ref, y_ref):
    h = p_ref[...]
    for r in range(R):
        x = jnp.concatenate([pb_ref[0, r], pb_ref[1, r]], axis=-1)
        x = x * inv_ref[r]
        h = h + jnp.dot(x, w2_ref[r][64:D], preferred_element_type=jnp.float32)
    h = jnp.maximum(h + b2_ref[...], 0.0)
    for r in range(R):
        y_ref[r] = jnp.dot(h, w3_ref[r], preferred_element_type=jnp.float32)


def _l1b_dense(pb, inv, w2, b2, w3, partial):
    return pl.pallas_call(
        _l1b_body,
        grid=(N // _NB,),
        in_specs=[
            pl.BlockSpec((NC, R, _NB, 32), lambda i: (0, 0, i, 0)),
            pl.BlockSpec((R, _NB, 1), lambda i: (0, i, 0)),
            pl.BlockSpec((R, D, D), lambda i: (0, 0, 0)),
            pl.BlockSpec((1, D), lambda i: (0, 0)),
            pl.BlockSpec((R, D, 16), lambda i: (0, 0, 0)),
            pl.BlockSpec((_NB, D), lambda i: (i, 0)),
        ],
        out_specs=pl.BlockSpec((R, _NB, 16), lambda i: (0, i, 0)),
        out_shape=jax.ShapeDtypeStruct((R, N, 16), jnp.float32),
    )(pb, inv, w2, b2, w3, partial)


def _out_body(accp_ref, inv_ref, b_ref, o_ref):
    y = jnp.zeros((_NB, 16), jnp.float32)
    for r in range(R):
        y = y + (accp_ref[0, r] + accp_ref[1, r]) * inv_ref[r]
    o_ref[...] = y + b_ref[...]


def _out_dense(accp, inv, b3):
    return pl.pallas_call(
        _out_body,
        grid=(N // _NB,),
        in_specs=[
            pl.BlockSpec((NC, R, _NB, 16), lambda i: (0, 0, i, 0)),
            pl.BlockSpec((R, _NB, 1), lambda i: (0, i, 0)),
            pl.BlockSpec((1, 16), lambda i: (0, 0)),
        ],
        out_specs=pl.BlockSpec((_NB, 16), lambda i: (i, 0)),
        out_shape=jax.ShapeDtypeStruct((N, 16), jnp.float32),
    )(accp, inv, b3)


def _aggregate(tabs, srci, dsti, zeros32):
    out = []
    for p in range(2):
        acc = _scatter_kernel(tabs[p].reshape(2 * N, 32), srci, dsti, zeros32)
        out.append(acc.reshape(NC, R, NPAD, 32))
    return out


def kernel(edge_src, edge_dst, edge_type, embed, h_bias1, W2, b2, W3, b3):
    pad = EPAD - E
    src_p = jnp.concatenate([edge_src, jnp.zeros((pad,), jnp.int32)])
    srci = jnp.stack([src_p, src_p + N]).reshape(NC, EROWS, LW)
    typ_p = jnp.concatenate([edge_type, jnp.zeros((pad,), jnp.int32)])
    srci2 = (typ_p * N + src_p).reshape(EROWS, LW)
    dst_p = jnp.concatenate(
        [edge_type * NPAD + edge_dst, jnp.full((pad,), N, jnp.int32)])
    dsti = dst_p.reshape(EROWS, LW)

    zeros32 = jnp.zeros((RNP, 32), jnp.float32)
    zeros8 = jnp.zeros((RNP, 8), jnp.float32)
    ones8 = jnp.ones((LW, 8), jnp.float32)

    degp = _deg_kernel(dsti, ones8, zeros8)
    degp = degp.reshape(NC, R, NPAD, 8)

    emb_tabs = [
        jnp.stack([embed[:, 64 * p:64 * p + 32],
                   embed[:, 64 * p + 32:64 * p + 64]]) for p in range(2)]
    accA = _scatter_kernel(emb_tabs[0].reshape(2 * N, 32), srci, dsti,
                           zeros32).reshape(NC, R, NPAD, 32)
    accB = _scatter_kernel(emb_tabs[1].reshape(2 * N, 32), srci, dsti,
                           zeros32).reshape(NC, R, NPAD, 32)
    b1 = h_bias1.reshape(1, D)
    ta = _l0_half(accA, degp, b1[:, 0:64], emit_inv=False)[0]
    tb, inv = _l0_half(accB, degp, b1[:, 64:D], emit_inv=True)

    acc1A = _scatter_kernel(ta.reshape(2 * N, 32), srci, dsti,
                            zeros32).reshape(NC, R, NPAD, 32)
    acc1B = _scatter_kernel(tb.reshape(2 * N, 32), srci, dsti,
                            zeros32).reshape(NC, R, NPAD, 32)
    partial = _l1a_dense(acc1A, inv, W2)
    y = _l1b_dense(acc1B, inv, W2, b2.reshape(1, D), W3, partial)

    zeros16 = jnp.zeros((RNP, 16), jnp.float32)
    accp = _scatter16_kernel(y.reshape(RN, 16), srci2, dsti, zeros16)
    accp = accp.reshape(NC, R, NPAD, 16)
    out = _out_dense(accp, inv, b3.reshape(1, 16))
    return out


# speedup vs baseline: 1.3165x; 1.0697x over previous
import functools

import jax
import jax.numpy as jnp
from jax import lax
from jax.experimental import pallas as pl
from jax.experimental.pallas import tpu as pltpu, tpu_sc as plsc

N = 10000
E = 320000
R = 4
RN = R * N
D = 128
NC = 2
NS = 16
LW = 128
EPAD = 327680
EROWS = EPAD // LW
ROWS_PER_SUB = EROWS // NS
ROWS_PER_WRK = EROWS // (NC * NS)
NPAD = 10240
RNP = R * NPAD
NSLICE = RNP // NS
NBUF = 2

_mesh = plsc.VectorSubcoreMesh(core_axis_name="c", subcore_axis_name="s")
_sc_params = pltpu.CompilerParams(use_tc_tiling_on_sc=False)


@functools.partial(
    pl.kernel,
    out_type=jax.ShapeDtypeStruct((NC, RNP, 8), jnp.float32),
    mesh=_mesh,
    scratch_types=[
        pltpu.VMEM((ROWS_PER_WRK, LW), jnp.int32),
        pltpu.VMEM((LW, 8), jnp.float32),
        pltpu.VMEM_SHARED((RNP, 8), jnp.float32),
        pltpu.SemaphoreType.DMA,
    ],
    compiler_params=_sc_params,
)
def _deg_kernel(dsti_hbm, ones_hbm, zeros_hbm, out_hbm, dsti_v, ones_v, acc_sh,
                sem):
    c = lax.axis_index("c")
    s = lax.axis_index("s")
    wid = s * NC + c
    pltpu.sync_copy(zeros_hbm.at[pl.ds(s * NSLICE, NSLICE)],
                    acc_sh.at[pl.ds(s * NSLICE, NSLICE)])
    pltpu.sync_copy(dsti_hbm.at[pl.ds(wid * ROWS_PER_WRK, ROWS_PER_WRK)], dsti_v)
    pltpu.sync_copy(ones_hbm, ones_v)
    plsc.subcore_barrier()

    def fire(j, carry):
        pltpu.async_copy(ones_v, acc_sh.at[dsti_v.at[j]], sem, add=True)
        return carry

    def drain(j, carry):
        pltpu.make_async_copy(ones_v, acc_sh.at[dsti_v.at[j]], sem).wait()
        return carry

    lax.fori_loop(0, ROWS_PER_WRK, fire, 0, unroll=False)
    lax.fori_loop(0, ROWS_PER_WRK, drain, 0, unroll=False)
    plsc.subcore_barrier()
    pltpu.sync_copy(acc_sh.at[pl.ds(s * NSLICE, NSLICE)],
                    out_hbm.at[c, pl.ds(s * NSLICE, NSLICE)])


def _make_scatter(w, split_cores, nbuf=NBUF):
    nrows = ROWS_PER_WRK if split_cores else ROWS_PER_SUB
    idx_halves = not split_cores
    srows = nrows // 2 if idx_halves else nrows
    scratch = [
        pltpu.VMEM((srows, LW), jnp.int32),
        pltpu.VMEM((nrows, LW), jnp.int32),
    ]
    scratch += [pltpu.VMEM((LW, w), jnp.float32) for _ in range(nbuf)]
    scratch += [pltpu.VMEM_SHARED((RNP, w), jnp.float32)]
    scratch += [pltpu.SemaphoreType.DMA for _ in range(2 * nbuf)]

    def body(table_hbm, srci_hbm, dsti_hbm, zeros_hbm, out_hbm,
             srci_v, dsti_v, *rest):
        bufs = rest[:nbuf]
        acc_sh = rest[nbuf]
        gsems = rest[nbuf + 1:nbuf + 1 + nbuf]
        ssems = rest[nbuf + 1 + nbuf:]
        c = lax.axis_index("c")
        s = lax.axis_index("s")
        zdesc = pltpu.async_copy(zeros_hbm.at[pl.ds(s * NSLICE, NSLICE)],
                                 acc_sh.at[pl.ds(s * NSLICE, NSLICE)],
                                 gsems[0])
        if split_cores:
            row0 = c * (EROWS // NC) + s * nrows
            sdesc = pltpu.async_copy(srci_hbm.at[pl.ds(row0, nrows)], srci_v,
                                     gsems[1])
            ddesc = pltpu.async_copy(dsti_hbm.at[pl.ds(row0, nrows)], dsti_v,
                                     ssems[0])
        else:
            row0 = s * nrows
            sdesc = pltpu.async_copy(srci_hbm.at[c, pl.ds(row0, srows)],
                                     srci_v, gsems[1])
            ddesc = pltpu.async_copy(dsti_hbm.at[pl.ds(row0, nrows)], dsti_v,
                                     ssems[0])
        zdesc.wait()
        sdesc.wait()
        ddesc.wait()
        plsc.subcore_barrier()

        def gather(j, jl, b):
            pltpu.async_copy(table_hbm.at[srci_v.at[jl]], bufs[b], gsems[b])

        def gwait(jl, b):
            pltpu.make_async_copy(table_hbm.at[srci_v.at[jl]], bufs[b],
                                  gsems[b]).wait()

        def scat(j, b):
            pltpu.async_copy(bufs[b], acc_sh.at[dsti_v.at[j]], ssems[b],
                             add=True)

        def swait(j, b):
            pltpu.make_async_copy(bufs[b], acc_sh.at[dsti_v.at[j]],
                                  ssems[b]).wait()

        def run_span(j0, span):
            for b in range(nbuf):
                gather(j0 + b, b, b)

            def step(k, carry):
                j = j0 + nbuf * k
                jl = nbuf * k
                for b in range(nbuf):
                    gwait(jl + b, b)
                    scat(j + b, b)
                for b in range(nbuf):
                    swait(j + b, b)

                    @pl.when(k < span // nbuf - 1)
                    def _(b=b, j=j, jl=jl):
                        gather(j + b + nbuf, jl + b + nbuf, b)

                return carry

            lax.fori_loop(0, span // nbuf, step, 0, unroll=False)

        if idx_halves:
            run_span(0, srows)
            pltpu.sync_copy(srci_hbm.at[c, pl.ds(row0 + srows, srows)], srci_v)
            run_span(srows, srows)
        else:
            run_span(0, nrows)
        plsc.subcore_barrier()
        pltpu.sync_copy(acc_sh.at[pl.ds(s * NSLICE, NSLICE)],
                        out_hbm.at[c, pl.ds(s * NSLICE, NSLICE)])

    return pl.kernel(
        body,
        out_type=jax.ShapeDtypeStruct((NC, RNP, w), jnp.float32),
        mesh=_mesh,
        scratch_types=scratch,
        compiler_params=_sc_params,
    )


_scatter_kernel = _make_scatter(32, split_cores=False, nbuf=4)
_scatter16_kernel = _make_scatter(16, split_cores=True, nbuf=8)


_NB = 1000


def _chunk_cols(pa_ref, pb_ref, r):
    return [pa_ref[0, r], pa_ref[1, r], pb_ref[0, r], pb_ref[1, r]]


def _half_body(emit_inv, ph_ref, degp_ref, b_ref, *outs):
    t_ref = outs[0]
    deg = degp_ref[0, :, :, 0:1] + degp_ref[1, :, :, 0:1]
    inv = 1.0 / jnp.maximum(deg, 1.0)
    for k in range(2):
        h = jnp.zeros((_NB, 32), jnp.float32)
        for r in range(R):
            h = h + ph_ref[k, r] * inv[r]
        t_ref[k] = jnp.maximum(h + b_ref[:, 32 * k:32 * k + 32], 0.0)
    if emit_inv:
        outs[1][...] = inv


def _l0_half(ph, degp, bh, emit_inv):
    out_specs = [pl.BlockSpec((2, _NB, 32), lambda i: (0, i, 0))]
    out_shape = [jax.ShapeDtypeStruct((2, N, 32), jnp.float32)]
    if emit_inv:
        out_specs.append(pl.BlockSpec((R, _NB, 1), lambda i: (0, i, 0)))
        out_shape.append(jax.ShapeDtypeStruct((R, N, 1), jnp.float32))
    return pl.pallas_call(
        functools.partial(_half_body, emit_inv),
        grid=(N // _NB,),
        in_specs=[
            pl.BlockSpec((NC, R, _NB, 32), lambda i: (0, 0, i, 0)),
            pl.BlockSpec((NC, R, _NB, 8), lambda i: (0, 0, i, 0)),
            pl.BlockSpec((1, 64), lambda i: (0, 0)),
        ],
        out_specs=out_specs,
        out_shape=out_shape,
    )(ph, degp, bh)


def _l1a_body(pa_ref, inv_ref, w2_ref, p_ref):
    p = jnp.zeros((_NB, D), jnp.float32)
    for r in range(R):
        x = jnp.concatenate([pa_ref[0, r], pa_ref[1, r]], axis=-1)
        x = x * inv_ref[r]
        p = p + jnp.dot(x, w2_ref[r][0:64], preferred_element_type=jnp.float32)
    p_ref[...] = p


def _l1a_dense(pa, inv, w2):
    return pl.pallas_call(
        _l1a_body,
        grid=(N // _NB,),
        in_specs=[
            pl.BlockSpec((NC, R, _NB, 32), lambda i: (0, 0, i, 0)),
            pl.BlockSpec((R, _NB, 1), lambda i: (0, i, 0)),
            pl.BlockSpec((R, D, D), lambda i: (0, 0, 0)),
        ],
        out_specs=pl.BlockSpec((_NB, D), lambda i: (i, 0)),
        out_shape=jax.ShapeDtypeStruct((N, D), jnp.float32),
    )(pa, inv, w2)


def _l1b_body(pb_ref, inv_ref, w2_ref, b2_ref, w3_ref, p_ref, y_ref):
    h = p_ref[...]
    for r in range(R):
        x = jnp.concatenate([pb_ref[0, r], pb_ref[1, r]], axis=-1)
        x = x * inv_ref[r]
        h = h + jnp.dot(x, w2_ref[r][64:D], preferred_element_type=jnp.float32)
    h = jnp.maximum(h + b2_ref[...], 0.0)
    for r in range(R):
        y_ref[r] = jnp.dot(h, w3_ref[r], preferred_element_type=jnp.float32)


def _l1b_dense(pb, inv, w2, b2, w3, partial):
    return pl.pallas_call(
        _l1b_body,
        grid=(N // _NB,),
        in_specs=[
            pl.BlockSpec((NC, R, _NB, 32), lambda i: (0, 0, i, 0)),
            pl.BlockSpec((R, _NB, 1), lambda i: (0, i, 0)),
            pl.BlockSpec((R, D, D), lambda i: (0, 0, 0)),
            pl.BlockSpec((1, D), lambda i: (0, 0)),
            pl.BlockSpec((R, D, 16), lambda i: (0, 0, 0)),
            pl.BlockSpec((_NB, D), lambda i: (i, 0)),
        ],
        out_specs=pl.BlockSpec((R, _NB, 16), lambda i: (0, i, 0)),
        out_shape=jax.ShapeDtypeStruct((R, N, 16), jnp.float32),
    )(pb, inv, w2, b2, w3, partial)


def _out_body(accp_ref, inv_ref, b_ref, o_ref):
    y = jnp.zeros((_NB, 16), jnp.float32)
    for r in range(R):
        y = y + (accp_ref[0, r] + accp_ref[1, r]) * inv_ref[r]
    o_ref[...] = y + b_ref[...]


def _out_dense(accp, inv, b3):
    return pl.pallas_call(
        _out_body,
        grid=(N // _NB,),
        in_specs=[
            pl.BlockSpec((NC, R, _NB, 16), lambda i: (0, 0, i, 0)),
            pl.BlockSpec((R, _NB, 1), lambda i: (0, i, 0)),
            pl.BlockSpec((1, 16), lambda i: (0, 0)),
        ],
        out_specs=pl.BlockSpec((_NB, 16), lambda i: (i, 0)),
        out_shape=jax.ShapeDtypeStruct((N, 16), jnp.float32),
    )(accp, inv, b3)


def _aggregate(tabs, srci, dsti, zeros32):
    out = []
    for p in range(2):
        acc = _scatter_kernel(tabs[p].reshape(2 * N, 32), srci, dsti, zeros32)
        out.append(acc.reshape(NC, R, NPAD, 32))
    return out


def kernel(edge_src, edge_dst, edge_type, embed, h_bias1, W2, b2, W3, b3):
    pad = EPAD - E
    src_p = jnp.concatenate([edge_src, jnp.zeros((pad,), jnp.int32)])
    srci = jnp.stack([src_p, src_p + N]).reshape(NC, EROWS, LW)
    typ_p = jnp.concatenate([edge_type, jnp.zeros((pad,), jnp.int32)])
    srci2 = (typ_p * N + src_p).reshape(EROWS, LW)
    dst_p = jnp.concatenate(
        [edge_type * NPAD + edge_dst, jnp.full((pad,), N, jnp.int32)])
    dsti = dst_p.reshape(EROWS, LW)

    zeros32 = jnp.zeros((RNP, 32), jnp.float32)
    zeros8 = jnp.zeros((RNP, 8), jnp.float32)
    ones8 = jnp.ones((LW, 8), jnp.float32)

    degp = _deg_kernel(dsti, ones8, zeros8)
    degp = degp.reshape(NC, R, NPAD, 8)

    emb_tabs = [
        jnp.stack([embed[:, 64 * p:64 * p + 32],
                   embed[:, 64 * p + 32:64 * p + 64]]) for p in range(2)]
    accA = _scatter_kernel(emb_tabs[0].reshape(2 * N, 32), srci, dsti,
                           zeros32).reshape(NC, R, NPAD, 32)
    accB = _scatter_kernel(emb_tabs[1].reshape(2 * N, 32), srci, dsti,
                           zeros32).reshape(NC, R, NPAD, 32)
    b1 = h_bias1.reshape(1, D)
    ta = _l0_half(accA, degp, b1[:, 0:64], emit_inv=False)[0]
    tb, inv = _l0_half(accB, degp, b1[:, 64:D], emit_inv=True)

    acc1A = _scatter_kernel(ta.reshape(2 * N, 32), srci, dsti,
                            zeros32).reshape(NC, R, NPAD, 32)
    acc1B = _scatter_kernel(tb.reshape(2 * N, 32), srci, dsti,
                            zeros32).reshape(NC, R, NPAD, 32)
    partial = _l1a_dense(acc1A, inv, W2)
    y = _l1b_dense(acc1B, inv, W2, b2.reshape(1, D), W3, partial)

    zeros16 = jnp.zeros((RNP, 16), jnp.float32)
    accp = _scatter16_kernel(y.reshape(RN, 16), srci2, dsti, zeros16)
    accp = accp.reshape(NC, R, NPAD, 16)
    out = _out_dense(accp, inv, b3.reshape(1, 16))
    return out
